# baseline (speedup 1.0000x reference)
import itertools

import jax
import jax.numpy as jnp
from jax.experimental import pallas as pl
from jax.experimental.pallas import tpu as pltpu


_EPS = 1e-5
_SLOPE = 0.2
_B = 8


def _lrelu(v):
    return jnp.maximum(v, _SLOPE * v)


_TAP = {0: (1, -1), 1: (0, 0), 2: (1, 0), 3: (0, 1)}


def _resolve(k, obits):
    p, c = _TAP[k]
    bits = [p]
    for ob in obits:
        e = ob + c
        b = e & 1
        bits.append(b)
        c = (e - b) >> 1
    return bits, c


def _shift_u(z, d):
    if d == -1:
        return jnp.concatenate([jnp.zeros_like(z[:, :1]), z[:, :-1]], axis=1)
    if d == 1:
        return jnp.concatenate([z[:, 1:], jnp.zeros_like(z[:, :1])], axis=1)
    return z


def _shift_vp(z, d):
    if d == -1:
        return jnp.concatenate([jnp.zeros_like(z[:, :, :1]), z[:, :, :-1]], axis=2)
    if d == 1:
        return jnp.concatenate([z[:, :, 1:], jnp.zeros_like(z[:, :, :1])], axis=2)
    return z


def _shift_packed(z, cc, Cin):
    if cc == 0:
        return z
    lo, hi = z[..., :Cin], z[..., Cin:]
    if cc == -1:
        return jnp.concatenate([_shift_vp(hi, -1), lo], axis=-1)
    return jnp.concatenate([hi, _shift_vp(lo, 1)], axis=-1)


def _conv_in_lrelu(y, B, Hin, Win, Lout, wbd, g, be, out_bf16):
    Lin = Lout + 1
    Cin2 = y.shape[-1]
    Cin = Cin2 // 2
    Cout2 = wbd.shape[-1]
    Cout = Cout2 // 2
    Hr, Wrp = Hin >> Lin, (Win >> Lin) >> 1
    A = y.reshape((B,) + (2, 2) * Lin + (Hr, Wrp, Cin2))
    blocks = []
    for obits in itertools.product((0, 1), repeat=2 * Lout):
        rbits_o, cbits_o = obits[0::2], obits[1::2]
        taps = []
        for kh in range(4):
            ibits_r, cr = _resolve(kh, rbits_o)
            for kw in range(4):
                ibits_c, cc = _resolve(kw, cbits_o)
                idx = tuple(b for pair in zip(ibits_r, ibits_c) for b in pair)
                piece = A[(slice(None),) + idx]
                taps.append(_shift_packed(_shift_u(piece, cr), cc, Cin))
        blocks.append(jnp.concatenate(taps, axis=-1))
    xim = jnp.stack(blocks, axis=1)
    Mp = (4 ** Lout) * Hr * Wrp
    xim = xim.reshape(B * Mp, 16 * Cin2)
    acc = jnp.dot(xim, wbd, preferred_element_type=jnp.float32)
    a = acc.reshape(B, Mp, Cout2)
    s1 = jnp.sum(a, axis=1, keepdims=True)
    s2 = jnp.sum(a * a, axis=1, keepdims=True)
    n = 2.0 * Mp
    mean = (s1[..., :Cout] + s1[..., Cout:]) / n
    var = (s2[..., :Cout] + s2[..., Cout:]) / n - mean * mean
    scale = g.reshape(1, 1, Cout) * jax.lax.rsqrt(var + _EPS)
    shift = be.reshape(1, 1, Cout) - mean * scale
    scale2 = jnp.concatenate([scale, scale], axis=-1)
    shift2 = jnp.concatenate([shift, shift], axis=-1)
    out = _lrelu(a * scale2 + shift2)
    if out_bf16:
        out = out.astype(jnp.bfloat16)
    return out.reshape(B * Mp, Cout2)


def _make_disc_kernel(B, H1, W1):
    M1p = H1 * W1 // 2

    def _disc_kernel(x1_ref, w1_ref, b1_ref, w2_ref, g2_ref, be2_ref,
                     w3_ref, g3_ref, be3_ref, w4_ref, g4_ref, be4_ref,
                     wfc_ref, bfc_ref, o_ref):
        y = jnp.dot(x1_ref[...].reshape(B * M1p, x1_ref.shape[-1]), w1_ref[...],
                    preferred_element_type=jnp.float32)
        y = _lrelu(y + b1_ref[...]).astype(jnp.bfloat16)

        y = _conv_in_lrelu(y, B, H1, W1, 2, w2_ref[...], g2_ref[...], be2_ref[...], True)
        y = _conv_in_lrelu(y, B, H1 // 2, W1 // 2, 1, w3_ref[...], g3_ref[...], be3_ref[...], True)
        y = _conv_in_lrelu(y, B, H1 // 4, W1 // 4, 0, w4_ref[...], g4_ref[...], be4_ref[...], False)

        M4p, C4p = wfc_ref.shape
        z = jnp.sum(y.reshape(B, M4p, C4p) * wfc_ref[...][None], axis=1)
        z = jnp.sum(z, axis=1, keepdims=True) + bfc_ref[...]
        o_ref[...] = 0.5 * (jnp.tanh(0.5 * z) + 1.0)

    return _disc_kernel


def _blockdiag2(w):
    K, C = w.shape
    z = jnp.zeros((2, K, 2, C), w.dtype)
    z = z.at[0, :, 0, :].set(w).at[1, :, 1, :].set(w)
    return z.reshape(2 * K, 2 * C)


def _blockdiag_taps(w):
    T, Cin, Cout = w.shape
    z = jnp.zeros((T, 2, Cin, 2, Cout), w.dtype)
    z = z.at[:, 0, :, 0, :].set(w).at[:, 1, :, 1, :].set(w)
    return z.reshape(T * 2 * Cin, 2 * Cout)


def kernel(w1, b1, t2, w2, g2, be2, t3, w3, g3, be3, t4, w4, g4, be4,
           w_fc, b_fc, x_nchw):
    del t2, t3, t4
    N, Cin, H, W = x_nchw.shape
    Ho, Wo = H // 2, W // 2
    M1 = Ho * Wo
    B = _B if N % _B == 0 else 1

    K1 = 16 * Cin
    eye = jnp.eye(K1, dtype=jnp.bfloat16).reshape(K1, 4, 4, Cin)
    eye = jnp.transpose(eye, (0, 3, 1, 2))
    x1 = jax.lax.conv_general_dilated(
        x_nchw.astype(jnp.bfloat16), eye, (2, 2), ((1, 1), (1, 1)),
        dimension_numbers=("NCHW", "OIHW", "NHWC"),
        preferred_element_type=jnp.bfloat16)
    Hr, Wr = Ho >> 3, Wo >> 3
    x1 = x1.reshape(N, Hr, 2, 2, 2, Wr, 2, 2, 2, K1)
    x1 = jnp.transpose(x1, (0, 4, 8, 3, 7, 2, 6, 1, 5, 9))
    x1 = x1.reshape(N, M1 // 2, 2 * K1)

    w1p = _blockdiag2(w1)
    b1p = jnp.concatenate([b1, b1], axis=-1)
    w2p = _blockdiag_taps(w2)
    w3p = _blockdiag_taps(w3)
    w4p = _blockdiag_taps(w4)
    M4, C4 = w_fc.shape
    wfc_p = w_fc.reshape(M4 // 2, 2 * C4)

    in_specs = [
        pl.BlockSpec((B, M1 // 2, 2 * K1), lambda n: (n, 0, 0)),
        pl.BlockSpec(w1p.shape, lambda n: (0, 0)),
        pl.BlockSpec(b1p.shape, lambda n: (0, 0)),
        pl.BlockSpec(w2p.shape, lambda n: (0, 0)),
        pl.BlockSpec(g2.shape, lambda n: (0, 0)),
        pl.BlockSpec(be2.shape, lambda n: (0, 0)),
        pl.BlockSpec(w3p.shape, lambda n: (0, 0)),
        pl.BlockSpec(g3.shape, lambda n: (0, 0)),
        pl.BlockSpec(be3.shape, lambda n: (0, 0)),
        pl.BlockSpec(w4p.shape, lambda n: (0, 0)),
        pl.BlockSpec(g4.shape, lambda n: (0, 0)),
        pl.BlockSpec(be4.shape, lambda n: (0, 0)),
        pl.BlockSpec(wfc_p.shape, lambda n: (0, 0)),
        pl.BlockSpec(b_fc.shape, lambda n: (0, 0)),
    ]
    out = pl.pallas_call(
        _make_disc_kernel(B, Ho, Wo),
        out_shape=jax.ShapeDtypeStruct((N, 1), jnp.float32),
        grid=(N // B,),
        in_specs=in_specs,
        out_specs=pl.BlockSpec((B, 1), lambda n: (n, 0)),
        compiler_params=pltpu.CompilerParams(
            dimension_semantics=("parallel",),
            vmem_limit_bytes=48 * 1024 * 1024),
    )(x1, w1p, b1p, w2p, g2, be2, w3p, g3, be3, w4p, g4, be4, wfc_p, b_fc)
    return out


# speedup vs baseline: 1.8742x; 1.8742x over previous
import itertools

import jax
import jax.numpy as jnp
from jax.experimental import pallas as pl
from jax.experimental.pallas import tpu as pltpu


_EPS = 1e-5
_SLOPE = 0.2
_B = 32


def _lrelu(v):
    return jnp.maximum(v, _SLOPE * v)


_TAP = {0: (1, -1), 1: (0, 0), 2: (1, 0), 3: (0, 1)}


def _resolve(k, obits):
    p, c = _TAP[k]
    bits = [p]
    for ob in obits:
        e = ob + c
        b = e & 1
        bits.append(b)
        c = (e - b) >> 1
    return bits, c


def _shift_u(z, d):
    if d == -1:
        return jnp.concatenate([jnp.zeros_like(z[:, :1]), z[:, :-1]], axis=1)
    if d == 1:
        return jnp.concatenate([z[:, 1:], jnp.zeros_like(z[:, :1])], axis=1)
    return z


def _shift_v(z, d):
    if d == -1:
        return jnp.concatenate([jnp.zeros_like(z[:, :, :1]), z[:, :, :-1]], axis=2)
    if d == 1:
        return jnp.concatenate([z[:, :, 1:], jnp.zeros_like(z[:, :, :1])], axis=2)
    return z


def _conv_in_lrelu(y, B, Hin, Win, Lout, w, g, be, out_bf16):
    Lin = Lout + 1
    Cin = y.shape[-1]
    Cout = w.shape[-1]
    Hr, Wr = Hin >> Lin, Win >> Lin
    A = y.reshape((B,) + (2, 2) * Lin + (Hr, Wr, Cin))
    accs = []
    for obits in itertools.product((0, 1), repeat=2 * Lout):
        rbits_o, cbits_o = obits[0::2], obits[1::2]
        taps = []
        for kh in range(4):
            ibits_r, cr = _resolve(kh, rbits_o)
            for kw in range(4):
                ibits_c, cc = _resolve(kw, cbits_o)
                idx = tuple(b for pair in zip(ibits_r, ibits_c) for b in pair)
                piece = A[(slice(None),) + idx]
                taps.append(_shift_v(_shift_u(piece, cr), cc))
        blk = jnp.concatenate(taps, axis=-1)
        accs.append(jnp.dot(blk.reshape(B * Hr * Wr, 16 * Cin), w,
                            preferred_element_type=jnp.float32))
    Mo = (4 ** Lout) * Hr * Wr
    acc = jnp.stack([a.reshape(B, Hr * Wr, Cout) for a in accs], axis=1)
    acc = acc.reshape(B * Mo, Cout)
    a = acc.reshape(B, Mo, Cout)
    mean = jnp.mean(a, axis=1, keepdims=True)
    var = jnp.mean(a * a, axis=1, keepdims=True) - mean * mean
    scale = g.reshape(1, 1, Cout) * jax.lax.rsqrt(var + _EPS)
    shift = be.reshape(1, 1, Cout) - mean * scale
    out = _lrelu(a * scale + shift)
    if out_bf16:
        out = out.astype(jnp.bfloat16)
    return out.reshape(B * Mo, Cout)


def _make_disc_kernel(B, H1, W1):
    M1 = H1 * W1

    def _disc_kernel(x1_ref, w1_ref, b1_ref, w2_ref, g2_ref, be2_ref,
                     w3_ref, g3_ref, be3_ref, w4_ref, g4_ref, be4_ref,
                     wfc_ref, bfc_ref, o_ref):
        y = jnp.dot(x1_ref[...].reshape(B * M1, x1_ref.shape[-1]), w1_ref[...],
                    preferred_element_type=jnp.float32)
        y = _lrelu(y + b1_ref[...]).astype(jnp.bfloat16)

        y = _conv_in_lrelu(y, B, H1, W1, 2, w2_ref[...], g2_ref[...], be2_ref[...], True)
        y = _conv_in_lrelu(y, B, H1 // 2, W1 // 2, 1, w3_ref[...], g3_ref[...], be3_ref[...], True)
        y = _conv_in_lrelu(y, B, H1 // 4, W1 // 4, 0, w4_ref[...], g4_ref[...], be4_ref[...], False)

        M4, C4 = wfc_ref.shape
        z = jnp.sum(y.reshape(B, M4, C4) * wfc_ref[...][None], axis=1)
        z = jnp.sum(z, axis=1, keepdims=True) + bfc_ref[...]
        o_ref[...] = 0.5 * (jnp.tanh(0.5 * z) + 1.0)

    return _disc_kernel


def kernel(w1, b1, t2, w2, g2, be2, t3, w3, g3, be3, t4, w4, g4, be4,
           w_fc, b_fc, x_nchw):
    del t2, t3, t4
    N, Cin, H, W = x_nchw.shape
    Ho, Wo = H // 2, W // 2
    M1 = Ho * Wo
    B = _B if N % _B == 0 else 1

    K1 = 16 * Cin
    eye = jnp.eye(K1, dtype=jnp.bfloat16).reshape(K1, 4, 4, Cin)
    eye = jnp.transpose(eye, (0, 3, 1, 2))
    x1 = jax.lax.conv_general_dilated(
        x_nchw.astype(jnp.bfloat16), eye, (2, 2), ((1, 1), (1, 1)),
        dimension_numbers=("NCHW", "OIHW", "NHWC"),
        preferred_element_type=jnp.bfloat16)
    Hr, Wr = Ho >> 3, Wo >> 3
    x1 = x1.reshape(N, Hr, 2, 2, 2, Wr, 2, 2, 2, K1)
    x1 = jnp.transpose(x1, (0, 4, 8, 3, 7, 2, 6, 1, 5, 9))
    x1 = x1.reshape(N, M1, K1)

    w2f = w2.reshape(-1, w2.shape[-1])
    w3f = w3.reshape(-1, w3.shape[-1])
    w4f = w4.reshape(-1, w4.shape[-1])

    in_specs = [
        pl.BlockSpec((B, M1, 16 * Cin), lambda n: (n, 0, 0)),
        pl.BlockSpec(w1.shape, lambda n: (0, 0)),
        pl.BlockSpec(b1.shape, lambda n: (0, 0)),
        pl.BlockSpec(w2f.shape, lambda n: (0, 0)),
        pl.BlockSpec(g2.shape, lambda n: (0, 0)),
        pl.BlockSpec(be2.shape, lambda n: (0, 0)),
        pl.BlockSpec(w3f.shape, lambda n: (0, 0)),
        pl.BlockSpec(g3.shape, lambda n: (0, 0)),
        pl.BlockSpec(be3.shape, lambda n: (0, 0)),
        pl.BlockSpec(w4f.shape, lambda n: (0, 0)),
        pl.BlockSpec(g4.shape, lambda n: (0, 0)),
        pl.BlockSpec(be4.shape, lambda n: (0, 0)),
        pl.BlockSpec(w_fc.shape, lambda n: (0, 0)),
        pl.BlockSpec(b_fc.shape, lambda n: (0, 0)),
    ]
    out = pl.pallas_call(
        _make_disc_kernel(B, Ho, Wo),
        out_shape=jax.ShapeDtypeStruct((N, 1), jnp.float32),
        grid=(N // B,),
        in_specs=in_specs,
        out_specs=pl.BlockSpec((B, 1), lambda n: (n, 0)),
        compiler_params=pltpu.CompilerParams(
            dimension_semantics=("parallel",),
            vmem_limit_bytes=48 * 1024 * 1024),
    )(x1, w1, b1, w2f, g2, be2, w3f, g3, be3, w4f, g4, be4, w_fc, b_fc)
    return out
